# Initial kernel scaffold; baseline (speedup 1.0000x reference)
import jax
import jax.numpy as jnp
from jax.experimental import pallas as pl


def kernel(x, table):
    raise NotImplementedError("write your pallas kernel here")


# speedup vs baseline: 1.4005x; 1.4005x over previous
import functools

import jax
import jax.numpy as jnp
from jax import lax
from jax.experimental import pallas as pl
from jax.experimental.pallas import tpu as pltpu
from jax.experimental.pallas import tpu_sc as plsc

B, L, D = 4096, 200, 32
NC, NS = 2, 16
NW = NC * NS
TOTAL = B * L
ROWS_PER_W = TOTAL // NW
SEQ_PER_CHUNK = 8
CHUNK = SEQ_PER_CHUNK * L
N_CHUNKS = ROWS_PER_W // CHUNK
UNROLL = 4


def _pe_table():
    pos = jnp.arange(L, dtype=jnp.float32).reshape(-1, 1)
    exponent = jnp.arange(0, D, 2, dtype=jnp.float32).reshape(1, -1) / D
    X = pos / jnp.power(10000.0, exponent)
    pe = jnp.zeros((L, D), dtype=jnp.float32)
    pe = pe.at[:, 0::2].set(jnp.sin(X))
    pe = pe.at[:, 1::2].set(jnp.cos(X))
    return pe


def _body(idx_hbm, table_hbm, pe_hbm, out_hbm, idx_v, rows_v, pe_v, sem):
    wid = lax.axis_index("s") * NC + lax.axis_index("c")
    base = wid * ROWS_PER_W
    pltpu.sync_copy(pe_hbm, pe_v)

    def chunk_body(c, carry):
        off = base + c * CHUNK
        pltpu.sync_copy(idx_hbm.at[pl.ds(off, CHUNK)], idx_v)
        pltpu.async_copy(table_hbm.at[idx_v], rows_v, sem).wait()

        def add_body(i, acc):
            r = i * UNROLL
            for u in range(UNROLL):
                for h in range(2):
                    v = pe_v[r + u, pl.ds(h * 16, 16)]
                    for s in range(SEQ_PER_CHUNK):
                        plsc.addupdate(
                            rows_v.at[s * L + r + u, pl.ds(h * 16, 16)], v)
            return acc

        lax.fori_loop(0, L // UNROLL, add_body, 0)
        pltpu.sync_copy(rows_v, out_hbm.at[pl.ds(off, CHUNK)])
        return carry

    lax.fori_loop(0, N_CHUNKS, chunk_body, 0)


@jax.jit
def kernel(x, table):
    idx = x.reshape(TOTAL).astype(jnp.int32)
    pe = _pe_table()
    mesh = plsc.VectorSubcoreMesh(core_axis_name="c", subcore_axis_name="s")
    out = pl.kernel(
        _body,
        out_type=jax.ShapeDtypeStruct((TOTAL, D), jnp.float32),
        mesh=mesh,
        scratch_types=[
            pltpu.VMEM((CHUNK,), jnp.int32),
            pltpu.VMEM((CHUNK, D), jnp.float32),
            pltpu.VMEM((L, D), jnp.float32),
            pltpu.SemaphoreType.DMA,
        ],
        compiler_params=pltpu.CompilerParams(use_tc_tiling_on_sc=False),
    )(idx, table, pe)
    return out.reshape(B, L, D)


# speedup vs baseline: 1.4851x; 1.0604x over previous
import jax
import jax.numpy as jnp
from jax import lax
from jax.experimental import pallas as pl
from jax.experimental.pallas import tpu as pltpu
from jax.experimental.pallas import tpu_sc as plsc

B, L, D = 4096, 200, 32
NC, NS = 2, 16
NW = NC * NS
TOTAL = B * L
ROWS_PER_W = TOTAL // NW
SEQ_PER_W = B // NW
SEQ_PER_CHUNK = 4
CHUNK = SEQ_PER_CHUNK * L
N_CHUNKS = ROWS_PER_W // CHUNK
NB = 3
UNROLL = 4


def _pe_table():
    pos = jnp.arange(L, dtype=jnp.float32).reshape(-1, 1)
    exponent = jnp.arange(0, D, 2, dtype=jnp.float32).reshape(1, -1) / D
    X = pos / jnp.power(10000.0, exponent)
    pe = jnp.zeros((L, D), dtype=jnp.float32)
    pe = pe.at[:, 0::2].set(jnp.sin(X))
    pe = pe.at[:, 1::2].set(jnp.cos(X))
    return pe


def _body(idx_hbm, table_hbm, pe_hbm, out_hbm,
          idx0, idx1, idx2, r0, r1, r2, pe_v,
          gs0, gs1, gs2, os0, os1, os2):
    idxv = [idx0, idx1, idx2]
    rows = [r0, r1, r2]
    gsem = [gs0, gs1, gs2]
    osem = [os0, os1, os2]
    wid = lax.axis_index("s") * NC + lax.axis_index("c")
    base = wid * ROWS_PER_W
    seq_base = wid * SEQ_PER_W
    pltpu.sync_copy(pe_hbm, pe_v)

    gd = [None] * NB
    wr = {}

    def stage_and_gather(c):
        b = c % NB
        pltpu.sync_copy(idx_hbm.at[pl.ds(base + c * CHUNK, CHUNK)], idxv[b])
        gd[b] = pltpu.async_copy(table_hbm.at[idxv[b]], rows[b], gsem[b])

    for c in range(min(NB, N_CHUNKS)):
        stage_and_gather(c)

    for c in range(N_CHUNKS):
        b = c % NB
        gd[b].wait()

        def add_body(i, acc, b=b):
            r = i * UNROLL
            for u in range(UNROLL):
                for h in range(2):
                    v = pe_v[r + u, pl.ds(h * 16, 16)]
                    for s in range(SEQ_PER_CHUNK):
                        plsc.addupdate(
                            rows[b].at[s * L + r + u, pl.ds(h * 16, 16)], v)
            return acc

        lax.fori_loop(0, L // UNROLL, add_body, 0)
        wr[c] = [
            pltpu.async_copy(
                rows[b].at[pl.ds(s * L, L)],
                out_hbm.at[seq_base + c * SEQ_PER_CHUNK + s],
                osem[b])
            for s in range(SEQ_PER_CHUNK)
        ]
        p, n = c - 1, c - 1 + NB
        if p >= 0 and n < N_CHUNKS:
            for d in wr.pop(p):
                d.wait()
            stage_and_gather(n)

    for c in sorted(wr):
        for d in wr[c]:
            d.wait()


@jax.jit
def kernel(x, table):
    idx = x.reshape(TOTAL).astype(jnp.int32)
    pe = _pe_table()
    mesh = plsc.VectorSubcoreMesh(core_axis_name="c", subcore_axis_name="s")
    out = pl.kernel(
        _body,
        out_type=jax.ShapeDtypeStruct((B, L, D), jnp.float32),
        mesh=mesh,
        scratch_types=[
            pltpu.VMEM((CHUNK,), jnp.int32),
            pltpu.VMEM((CHUNK,), jnp.int32),
            pltpu.VMEM((CHUNK,), jnp.int32),
            pltpu.VMEM((CHUNK, D), jnp.float32),
            pltpu.VMEM((CHUNK, D), jnp.float32),
            pltpu.VMEM((CHUNK, D), jnp.float32),
            pltpu.VMEM((L, D), jnp.float32),
            pltpu.SemaphoreType.DMA,
            pltpu.SemaphoreType.DMA,
            pltpu.SemaphoreType.DMA,
            pltpu.SemaphoreType.DMA,
            pltpu.SemaphoreType.DMA,
            pltpu.SemaphoreType.DMA,
        ],
        compiler_params=pltpu.CompilerParams(use_tc_tiling_on_sc=False),
    )(idx, table, pe)
    return out
